# baseline (speedup 1.0000x reference)
import functools

import jax
import jax.numpy as jnp
from jax import lax
from jax.experimental import pallas as pl
from jax.experimental.pallas import tpu as pltpu
from jax.experimental.pallas import tpu_sc as plsc

N_TRAIN = 100000
D = 128
RES = 2048
Q = 16
M = 32
ALPHA = 0.1

NC = 2
NS = 16
NW = NC * NS

N_TC = 44000
TCB = 11000
NTCB = N_TC // TCB
N_SC = N_TRAIN - N_TC
SC_BASE = N_TC
STRIDE = 1744
RPW = 1920
CH = 160
NCHUNK = RPW // CH
GPC = CH // 16

S1R, S1C = NTCB, TCB
S2R, S2C = 4, N_SC // 4
BIG = 3.0e38
IBIG = 2**31 - 1


def _sc_dist(x_hbm, q2b_hbm, d_hbm, buf_a, buf_b, qb_ref, dbuf_ref,
             sem_a, sem_b, sem_q, sem_d):
    wid = lax.axis_index("s") * NC + lax.axis_index("c")
    base = jnp.where(wid == NW - 1, N_TRAIN - RPW, SC_BASE + wid * STRIDE)

    pltpu.make_async_copy(q2b_hbm, qb_ref, sem_q).start()

    def copy(ci, buf, sem):
        return pltpu.make_async_copy(
            x_hbm.at[pl.ds((base + ci * CH) * D, CH * D)], buf, sem)

    copy(0, buf_a, sem_a).start()
    pltpu.make_async_copy(q2b_hbm, qb_ref, sem_q).wait()

    lanes = lax.iota(jnp.int32, 16)
    q2 = [qb_ref[pl.ds(16 * kk, 16)] for kk in range(8)]

    def do_chunk(ci, buf):
        def g_body(g, carry):
            dv = jnp.zeros((16,), jnp.float32)
            rb = g * (16 * D)
            for rr in range(16):
                acc = None
                for kk in range(8):
                    xv = buf[pl.ds(rb + rr * D + kk * 16, 16)]
                    t = xv * (xv - q2[kk])
                    acc = t if acc is None else acc + t
                s = jnp.sum(acc)
                dv = jnp.where(lanes == rr, s, dv)
            dbuf_ref[pl.ds(ci * CH + g * 16, 16)] = dv
            return carry

        lax.fori_loop(0, GPC, g_body, 0)

    def pair(i0, carry):
        ci = 2 * i0
        copy(ci, buf_a, sem_a).wait()
        copy(ci + 1, buf_b, sem_b).start()
        do_chunk(ci, buf_a)
        copy(ci + 1, buf_b, sem_b).wait()

        @pl.when(ci + 2 < NCHUNK)
        def _():
            copy(ci + 2, buf_a, sem_a).start()

        do_chunk(ci + 1, buf_b)
        return carry

    lax.fori_loop(0, NCHUNK // 2, pair, 0)

    ob = base - SC_BASE
    pltpu.make_async_copy(dbuf_ref, d_hbm.at[pl.ds(ob, RPW)], sem_d).start()
    pltpu.make_async_copy(dbuf_ref, d_hbm.at[pl.ds(ob, RPW)], sem_d).wait()


def _tc_dist(x_blk, newx_ref, d_out):
    hi = lax.Precision.HIGHEST
    xb = x_blk[...]
    q0 = newx_ref[0:1, :]
    ones = jnp.ones((1, D), jnp.float32)
    sq = lax.dot_general(ones, xb * xb, (((1,), (1,)), ((), ())),
                         precision=hi)
    dq = lax.dot_general(q0, xb, (((1,), (1,)), ((), ())), precision=hi)
    d_out[...] = (sq - 2.0 * dq)[None]


def _tc_finish(d1_in, d2_in, newx_ref, b0_ref, cw_ref, x_any, y_any,
               out_ref, s1_ref, s2_ref, xm_ref, ym_ref, idx_ref,
               sem_x, sem_y):
    hi = lax.Precision.HIGHEST

    s1_ref[...] = d1_in[...]
    s2_ref[...] = d2_in[...]
    flat1 = (lax.broadcasted_iota(jnp.int32, (S1R, S1C), 0) * S1C
             + lax.broadcasted_iota(jnp.int32, (S1R, S1C), 1))
    flat2 = (lax.broadcasted_iota(jnp.int32, (S2R, S2C), 0) * S2C
             + lax.broadcasted_iota(jnp.int32, (S2R, S2C), 1) + N_TC)

    def select(t, carry):
        s1 = s1_ref[...]
        s2 = s2_ref[...]
        m = jnp.minimum(jnp.min(s1), jnp.min(s2))
        fi = jnp.minimum(
            jnp.min(jnp.where(s1 == m, flat1, IBIG)),
            jnp.min(jnp.where(s2 == m, flat2, IBIG)))
        s1_ref[...] = jnp.where(flat1 == fi, BIG, s1)
        s2_ref[...] = jnp.where(flat2 == fi, BIG, s2)
        idx_ref[t] = fi
        return carry

    lax.fori_loop(0, M, select, 0, unroll=False)

    def fire(t, carry):
        r = idx_ref[t]
        pltpu.make_async_copy(
            x_any.at[pl.ds(r, 1)], xm_ref.at[pl.ds(t, 1)], sem_x).start()
        pltpu.make_async_copy(
            y_any.at[pl.ds(r, 1)], ym_ref.at[pl.ds(t, 1)], sem_y).start()
        return carry

    def drain(t, carry):
        r = idx_ref[t]
        pltpu.make_async_copy(
            x_any.at[pl.ds(r, 1)], xm_ref.at[pl.ds(t, 1)], sem_x).wait()
        pltpu.make_async_copy(
            y_any.at[pl.ds(r, 1)], ym_ref.at[pl.ds(t, 1)], sem_y).wait()
        return carry

    lax.fori_loop(0, M, fire, 0, unroll=False)
    lax.fori_loop(0, M, drain, 0, unroll=False)

    xm = xm_ref[...]
    ym = ym_ref[...]
    nx = newx_ref[...]
    b0 = b0_ref[...]
    cw = cw_ref[...]

    def act(z):
        return (jnp.exp(-z * z) + jnp.maximum(z, 0.0) + jnp.tanh(z)) / 3.0

    zm = b0 + lax.dot_general(cw, xm, (((1,), (1,)), ((), ())), precision=hi)
    zq = b0 + lax.dot_general(cw, nx, (((1,), (1,)), ((), ())), precision=hi)
    f = act(zm)
    hn = act(zq)

    xmean = jnp.mean(f, axis=1, keepdims=True)
    fc = f - xmean
    ymean = jnp.mean(ym, axis=0, keepdims=True)
    yc = ym - ymean

    g = lax.dot_general(fc, fc, (((0,), (0,)), ((), ())), precision=hi)
    ri = lax.broadcasted_iota(jnp.int32, (M, M), 0)
    ci = lax.broadcasted_iota(jnp.int32, (M, M), 1)
    g = g + jnp.where(ri == ci, jnp.float32(ALPHA), 0.0)

    W = M + D + M
    aug = jnp.concatenate(
        [g, yc, jnp.where(ri == ci, 1.0, 0.0).astype(jnp.float32)], axis=1)
    arow = lax.broadcasted_iota(jnp.int32, (M, W), 0)
    acol = lax.broadcasted_iota(jnp.int32, (M, W), 1)

    def gj(kk, m_):
        p = jnp.sum(jnp.where((arow == kk) & (acol == kk), m_, 0.0))
        colk = jnp.sum(jnp.where(acol == kk, m_, 0.0), axis=1, keepdims=True)
        rowk = jnp.sum(jnp.where(arow == kk, m_, 0.0), axis=0,
                       keepdims=True) / p
        m_ = m_ - colk * rowk
        return jnp.where(arow == kk, rowk, m_)

    aug = lax.fori_loop(0, M, gj, aug)
    beta = aug[:, M:M + D]
    ginv = aug[:, M + D:]
    resid = yc - lax.dot_general(g, beta, (((1,), (0,)), ((), ())),
                                 precision=hi)
    beta = beta + lax.dot_general(ginv, resid, (((1,), (0,)), ((), ())),
                                  precision=hi)

    u = lax.dot_general(hn, fc, (((0,), (0,)), ((), ())), precision=hi)
    v = lax.dot_general(xmean, fc, (((0,), (0,)), ((), ())), precision=hi)
    pred = lax.dot_general(u - v, beta, (((1,), (0,)), ((), ())),
                           precision=hi)
    out_ref[...] = pred + ymean


def kernel(x, y, new_x, bias, C, k):
    del k

    q2b = 2.0 * new_x[0]
    b0 = bias + C[:, 0:1]
    cw = C[:, 1:]

    mesh = plsc.VectorSubcoreMesh(core_axis_name="c", subcore_axis_name="s")
    sc_dist = functools.partial(
        pl.kernel,
        mesh=mesh,
        compiler_params=pltpu.CompilerParams(needs_layout_passes=False),
        out_type=jax.ShapeDtypeStruct((N_SC,), jnp.float32),
        scratch_types=[
            pltpu.VMEM((CH * D,), jnp.float32),
            pltpu.VMEM((CH * D,), jnp.float32),
            pltpu.VMEM((D,), jnp.float32),
            pltpu.VMEM((RPW,), jnp.float32),
            pltpu.SemaphoreType.DMA,
            pltpu.SemaphoreType.DMA,
            pltpu.SemaphoreType.DMA,
            pltpu.SemaphoreType.DMA,
        ],
    )(_sc_dist)
    d_sc = sc_dist(x.reshape(-1), q2b)

    d_tc = pl.pallas_call(
        _tc_dist,
        grid=(NTCB,),
        in_specs=[
            pl.BlockSpec((TCB, D), lambda i: (i, 0)),
            pl.BlockSpec((Q, D), lambda i: (0, 0)),
        ],
        out_specs=pl.BlockSpec((1, 1, TCB), lambda i: (i, 0, 0)),
        out_shape=jax.ShapeDtypeStruct((NTCB, 1, TCB), jnp.float32),
    )(x, new_x)

    pred = pl.pallas_call(
        _tc_finish,
        in_specs=[
            pl.BlockSpec((S1R, S1C), lambda: (0, 0)),
            pl.BlockSpec((S2R, S2C), lambda: (0, 0)),
            pl.BlockSpec((Q, D), lambda: (0, 0)),
            pl.BlockSpec((RES, 1), lambda: (0, 0)),
            pl.BlockSpec((RES, D), lambda: (0, 0)),
            pl.BlockSpec(memory_space=pl.ANY),
            pl.BlockSpec(memory_space=pl.ANY),
        ],
        out_specs=pl.BlockSpec((Q, D), lambda: (0, 0)),
        out_shape=jax.ShapeDtypeStruct((Q, D), jnp.float32),
        scratch_shapes=[
            pltpu.VMEM((S1R, S1C), jnp.float32),
            pltpu.VMEM((S2R, S2C), jnp.float32),
            pltpu.VMEM((M, D), jnp.float32),
            pltpu.VMEM((M, D), jnp.float32),
            pltpu.SMEM((M,), jnp.int32),
            pltpu.SemaphoreType.DMA,
            pltpu.SemaphoreType.DMA,
        ],
    )(d_tc.reshape(S1R, S1C), d_sc.reshape(S2R, S2C), new_x, b0, cw, x, y)

    return pred


# speedup vs baseline: 1.1250x; 1.1250x over previous
import functools

import jax
import jax.numpy as jnp
from jax import lax
from jax.experimental import pallas as pl
from jax.experimental.pallas import tpu as pltpu
from jax.experimental.pallas import tpu_sc as plsc

N_TRAIN = 100000
D = 128
RES = 2048
Q = 16
M = 32
ALPHA = 0.1

NC = 2
NS = 16
NW = NC * NS
STRIDE = 3120
RPW = 3200
CH = 160
NCHUNK = RPW // CH
GPC = CH // 16

SROW = 8
SCOL = N_TRAIN // SROW
BIG = 3.0e38
IBIG = 2**31 - 1


def _sc_dist(x_hbm, q2b_hbm, d_hbm, buf_a, buf_b, qb_ref, dbuf_ref,
             sem_a, sem_b, sem_q, sem_d):
    wid = lax.axis_index("s") * NC + lax.axis_index("c")
    base = jnp.where(wid == NW - 1, N_TRAIN - RPW, wid * STRIDE)

    pltpu.make_async_copy(q2b_hbm, qb_ref, sem_q).start()

    def copy(ci, buf, sem):
        return pltpu.make_async_copy(
            x_hbm.at[pl.ds((base + ci * CH) * D, CH * D)], buf, sem)

    copy(0, buf_a, sem_a).start()
    pltpu.make_async_copy(q2b_hbm, qb_ref, sem_q).wait()

    lanes = lax.iota(jnp.int32, 16)
    q2 = [qb_ref[pl.ds(16 * kk, 16)] for kk in range(8)]

    def do_chunk(ci, buf):
        def g_body(g, carry):
            dv = jnp.zeros((16,), jnp.float32)
            rb = g * (16 * D)
            for rr in range(16):
                acc = None
                for kk in range(8):
                    xv = buf[pl.ds(rb + rr * D + kk * 16, 16)]
                    t = xv * (xv - q2[kk])
                    acc = t if acc is None else acc + t
                s = jnp.sum(acc)
                dv = jnp.where(lanes == rr, s, dv)
            dbuf_ref[pl.ds(ci * CH + g * 16, 16)] = dv
            return carry

        lax.fori_loop(0, GPC, g_body, 0)

    def pair(i0, carry):
        ci = 2 * i0
        copy(ci, buf_a, sem_a).wait()
        copy(ci + 1, buf_b, sem_b).start()
        do_chunk(ci, buf_a)
        copy(ci + 1, buf_b, sem_b).wait()

        @pl.when(ci + 2 < NCHUNK)
        def _():
            copy(ci + 2, buf_a, sem_a).start()

        do_chunk(ci + 1, buf_b)
        return carry

    lax.fori_loop(0, NCHUNK // 2, pair, 0)

    pltpu.make_async_copy(dbuf_ref, d_hbm.at[pl.ds(base, RPW)], sem_d).start()
    pltpu.make_async_copy(dbuf_ref, d_hbm.at[pl.ds(base, RPW)], sem_d).wait()


def _tc_finish(d_in, newx_ref, b0_ref, cw_ref, x_any, y_any,
               out_ref, s_ref, xm_ref, ym_ref, idx_ref, sem_x, sem_y):
    hi = lax.Precision.HIGHEST

    s_ref[...] = d_in[...]
    rows = lax.broadcasted_iota(jnp.int32, (SROW, SCOL), 0)
    cols = lax.broadcasted_iota(jnp.int32, (SROW, SCOL), 1)
    flat = rows * SCOL + cols

    def select(t, carry):
        s = s_ref[...]
        m = jnp.min(s)
        fi = jnp.min(jnp.where(s == m, flat, IBIG))
        s_ref[...] = jnp.where(flat == fi, BIG, s)
        idx_ref[t] = fi
        return carry

    lax.fori_loop(0, M, select, 0, unroll=False)

    def fire(t, carry):
        r = idx_ref[t]
        pltpu.make_async_copy(
            x_any.at[pl.ds(r, 1)], xm_ref.at[pl.ds(t, 1)], sem_x).start()
        pltpu.make_async_copy(
            y_any.at[pl.ds(r, 1)], ym_ref.at[pl.ds(t, 1)], sem_y).start()
        return carry

    def drain(t, carry):
        r = idx_ref[t]
        pltpu.make_async_copy(
            x_any.at[pl.ds(r, 1)], xm_ref.at[pl.ds(t, 1)], sem_x).wait()
        pltpu.make_async_copy(
            y_any.at[pl.ds(r, 1)], ym_ref.at[pl.ds(t, 1)], sem_y).wait()
        return carry

    lax.fori_loop(0, M, fire, 0, unroll=False)
    lax.fori_loop(0, M, drain, 0, unroll=False)

    xm = xm_ref[...]
    ym = ym_ref[...]
    nx = newx_ref[...]
    b0 = b0_ref[...]
    cw = cw_ref[...]

    def act(z):
        return (jnp.exp(-z * z) + jnp.maximum(z, 0.0) + jnp.tanh(z)) / 3.0

    zm = b0 + lax.dot_general(cw, xm, (((1,), (1,)), ((), ())), precision=hi)
    zq = b0 + lax.dot_general(cw, nx, (((1,), (1,)), ((), ())), precision=hi)
    f = act(zm)
    hn = act(zq)

    xmean = jnp.mean(f, axis=1, keepdims=True)
    fc = f - xmean
    ymean = jnp.mean(ym, axis=0, keepdims=True)
    yc = ym - ymean

    g = lax.dot_general(fc, fc, (((0,), (0,)), ((), ())), precision=hi)
    ri = lax.broadcasted_iota(jnp.int32, (M, M), 0)
    ci = lax.broadcasted_iota(jnp.int32, (M, M), 1)
    g = g + jnp.where(ri == ci, jnp.float32(ALPHA), 0.0)

    W = M + D + M
    aug = jnp.concatenate(
        [g, yc, jnp.where(ri == ci, 1.0, 0.0).astype(jnp.float32)], axis=1)
    arow = lax.broadcasted_iota(jnp.int32, (M, W), 0)
    acol = lax.broadcasted_iota(jnp.int32, (M, W), 1)

    def gj(kk, m_):
        p = jnp.sum(jnp.where((arow == kk) & (acol == kk), m_, 0.0))
        colk = jnp.sum(jnp.where(acol == kk, m_, 0.0), axis=1, keepdims=True)
        rowk = jnp.sum(jnp.where(arow == kk, m_, 0.0), axis=0,
                       keepdims=True) / p
        m_ = m_ - colk * rowk
        return jnp.where(arow == kk, rowk, m_)

    aug = lax.fori_loop(0, M, gj, aug)
    beta = aug[:, M:M + D]
    ginv = aug[:, M + D:]
    resid = yc - lax.dot_general(g, beta, (((1,), (0,)), ((), ())),
                                 precision=hi)
    beta = beta + lax.dot_general(ginv, resid, (((1,), (0,)), ((), ())),
                                  precision=hi)

    u = lax.dot_general(hn, fc, (((0,), (0,)), ((), ())), precision=hi)
    v = lax.dot_general(xmean, fc, (((0,), (0,)), ((), ())), precision=hi)
    pred = lax.dot_general(u - v, beta, (((1,), (0,)), ((), ())),
                           precision=hi)
    out_ref[...] = pred + ymean


def kernel(x, y, new_x, bias, C, k):
    del k

    q2b = 2.0 * new_x[0]
    b0 = bias + C[:, 0:1]
    cw = C[:, 1:]

    mesh = plsc.VectorSubcoreMesh(core_axis_name="c", subcore_axis_name="s")
    sc_dist = functools.partial(
        pl.kernel,
        mesh=mesh,
        compiler_params=pltpu.CompilerParams(needs_layout_passes=False),
        out_type=jax.ShapeDtypeStruct((N_TRAIN,), jnp.float32),
        scratch_types=[
            pltpu.VMEM((CH * D,), jnp.float32),
            pltpu.VMEM((CH * D,), jnp.float32),
            pltpu.VMEM((D,), jnp.float32),
            pltpu.VMEM((RPW,), jnp.float32),
            pltpu.SemaphoreType.DMA,
            pltpu.SemaphoreType.DMA,
            pltpu.SemaphoreType.DMA,
            pltpu.SemaphoreType.DMA,
        ],
    )(_sc_dist)
    d = sc_dist(x.reshape(-1), q2b)

    pred = pl.pallas_call(
        _tc_finish,
        in_specs=[
            pl.BlockSpec((SROW, SCOL), lambda: (0, 0)),
            pl.BlockSpec((Q, D), lambda: (0, 0)),
            pl.BlockSpec((RES, 1), lambda: (0, 0)),
            pl.BlockSpec((RES, D), lambda: (0, 0)),
            pl.BlockSpec(memory_space=pl.ANY),
            pl.BlockSpec(memory_space=pl.ANY),
        ],
        out_specs=pl.BlockSpec((Q, D), lambda: (0, 0)),
        out_shape=jax.ShapeDtypeStruct((Q, D), jnp.float32),
        scratch_shapes=[
            pltpu.VMEM((SROW, SCOL), jnp.float32),
            pltpu.VMEM((M, D), jnp.float32),
            pltpu.VMEM((M, D), jnp.float32),
            pltpu.SMEM((M,), jnp.int32),
            pltpu.SemaphoreType.DMA,
            pltpu.SemaphoreType.DMA,
        ],
    )(d.reshape(SROW, SCOL), new_x, b0, cw, x, y)

    return pred
